# Initial kernel scaffold; baseline (speedup 1.0000x reference)
import jax
import jax.numpy as jnp
from jax.experimental import pallas as pl


def kernel(actions, global_idxes, h1_w, d2e_w, w_ih, w_hh, b_ih, b_hh, e2d_w, e2d_b):
    raise NotImplementedError("write your pallas kernel here")


# speedup vs baseline: 9.1187x; 9.1187x over previous
import jax
import jax.numpy as jnp
from jax.experimental import pallas as pl
from jax.experimental.pallas import tpu as pltpu

_T = 512


def _speaker_body(actions_ref, noise_ref, h1_ref, d2e_ref, wih_t_ref, whh_t_ref,
                  bih_ref, bhh_ref, e2d_t_ref, e2db_ref,
                  utt_ref, nouter_ref, logp_ref,
                  state_scr, tok_scr, alive_scr, nouter_scr, logp_scr):
    B, E = state_scr.shape
    V = e2db_ref.shape[1]
    t = pl.program_id(0)

    @pl.when(t == 0)
    def _init():
        for b in range(B):
            a = actions_ref[b]
            state_scr[pl.ds(b, 1), :] = h1_ref[pl.ds(a, 1), :]
        tok_scr[...] = jnp.zeros((B, 1), jnp.int32)
        alive_scr[...] = jnp.ones((B, 1), jnp.int32)
        nouter_scr[...] = jnp.full((B, 1), _T, jnp.int32)
        logp_scr[...] = jnp.zeros((B, 1), jnp.float32)

    embs = []
    for b in range(B):
        tb = tok_scr[b, 0]
        embs.append(d2e_ref[pl.ds(tb, 1), :])
    emb = jnp.concatenate(embs, axis=0)

    state = state_scr[...]
    gi = jax.lax.dot_general(emb, wih_t_ref[...], (((1,), (0,)), ((), ()))) + bih_ref[...]
    gh = jax.lax.dot_general(state, whh_t_ref[...], (((1,), (0,)), ((), ()))) + bhh_ref[...]
    r = jax.nn.sigmoid(gi[:, :E] + gh[:, :E])
    z = jax.nn.sigmoid(gi[:, E:2 * E] + gh[:, E:2 * E])
    n = jnp.tanh(gi[:, 2 * E:] + r * gh[:, 2 * E:])
    new_state = (1.0 - z) * n + z * state

    logits = jax.lax.dot_general(new_state, e2d_t_ref[...], (((1,), (0,)), ((), ()))) + e2db_ref[...]
    y = logits + noise_ref[0]
    m = jnp.max(y, axis=-1, keepdims=True)
    iota = jax.lax.broadcasted_iota(jnp.int32, (B, V), 1)
    tokv = jnp.min(jnp.where(y == m, iota, V), axis=-1, keepdims=True)

    xmax = jnp.max(logits, axis=-1, keepdims=True)
    shifted = logits - xmax
    lse = jnp.log(jnp.sum(jnp.exp(shifted), axis=-1, keepdims=True))
    sel = jnp.sum(jnp.where(iota == tokv, shifted, 0.0), axis=-1, keepdims=True)
    lp = sel - lse

    alive = alive_scr[...] > 0
    utt_ref[0] = jnp.where(alive, tokv, 0)
    logp_scr[...] = logp_scr[...] + jnp.where(alive, lp, 0.0)
    nouter_scr[...] = jnp.where(alive & (tokv == 0), t + 1, nouter_scr[...])
    alive_scr[...] = jnp.where(alive & (tokv != 0), 1, 0)
    state_scr[...] = jnp.where(alive, new_state, state)
    tok_scr[...] = jnp.where(alive, tokv, tok_scr[...])

    @pl.when(t == _T - 1)
    def _fin():
        nouter_ref[...] = nouter_scr[...]
        logp_ref[...] = logp_scr[...]


def kernel(actions, global_idxes, h1_w, d2e_w, w_ih, w_hh, b_ih, b_hh, e2d_w, e2d_b):
    del global_idxes
    B = actions.shape[0]
    A, E = h1_w.shape
    V, _ = d2e_w.shape
    T = _T

    skey = jax.random.key(42)
    keys = jax.vmap(lambda t_: jax.random.fold_in(skey, t_))(jnp.arange(T))
    noise = jax.vmap(lambda k: jax.random.gumbel(k, (B, V), jnp.float32))(keys)

    grid = (T,)
    out = pl.pallas_call(
        _speaker_body,
        grid=grid,
        in_specs=[
            pl.BlockSpec(memory_space=pltpu.SMEM),
            pl.BlockSpec((1, B, V), lambda t: (t, 0, 0)),
            pl.BlockSpec((A, E), lambda t: (0, 0)),
            pl.BlockSpec((V, E), lambda t: (0, 0)),
            pl.BlockSpec((E, 3 * E), lambda t: (0, 0)),
            pl.BlockSpec((E, 3 * E), lambda t: (0, 0)),
            pl.BlockSpec((1, 3 * E), lambda t: (0, 0)),
            pl.BlockSpec((1, 3 * E), lambda t: (0, 0)),
            pl.BlockSpec((E, V), lambda t: (0, 0)),
            pl.BlockSpec((1, V), lambda t: (0, 0)),
        ],
        out_specs=[
            pl.BlockSpec((1, B, 1), lambda t: (t, 0, 0)),
            pl.BlockSpec((B, 1), lambda t: (0, 0)),
            pl.BlockSpec((B, 1), lambda t: (0, 0)),
        ],
        out_shape=[
            jax.ShapeDtypeStruct((T, B, 1), jnp.int32),
            jax.ShapeDtypeStruct((B, 1), jnp.int32),
            jax.ShapeDtypeStruct((B, 1), jnp.float32),
        ],
        scratch_shapes=[
            pltpu.VMEM((B, E), jnp.float32),
            pltpu.VMEM((B, 1), jnp.int32),
            pltpu.VMEM((B, 1), jnp.int32),
            pltpu.VMEM((B, 1), jnp.int32),
            pltpu.VMEM((B, 1), jnp.float32),
        ],
        compiler_params=pltpu.CompilerParams(
            dimension_semantics=("arbitrary",),
            vmem_limit_bytes=100 * 1024 * 1024,
        ),
    )(actions, noise, h1_w, d2e_w, w_ih.T, w_hh.T,
      b_ih.reshape(1, 3 * E), b_hh.reshape(1, 3 * E), e2d_w.T, e2d_b.reshape(1, V))

    utt, nouter, logp = out
    utterance = utt.reshape(T, B).T
    return utterance, nouter.reshape(B), logp.reshape(B)
